# Initial kernel scaffold; baseline (speedup 1.0000x reference)
import jax
import jax.numpy as jnp
from jax.experimental import pallas as pl


def kernel(xyz):
    raise NotImplementedError("write your pallas kernel here")


# speedup vs baseline: 1.6365x; 1.6365x over previous
import functools

import jax
import jax.numpy as jnp
from jax import lax
from jax.experimental import pallas as pl
from jax.experimental.pallas import tpu as pltpu

_G = 1024
_K = 32
_INF = 3.0e38


def _fps_body(xyzT_ref, idx_ref, dist_scratch, far_scratch):
    B = xyzT_ref.shape[1]
    N = xyzT_ref.shape[2]
    col = lax.broadcasted_iota(jnp.int32, (B, N), 1)
    gcol = lax.broadcasted_iota(jnp.int32, (B, _G), 1)
    dist_scratch[...] = jnp.full((B, N), 1e10, jnp.float32)
    far_scratch[...] = jnp.zeros((B, 1), jnp.int32)
    idx_ref[...] = jnp.zeros((B, _G), jnp.int32)

    def body(i, _):
        farthest = far_scratch[...]
        idx_ref[...] = jnp.where(gcol == i, farthest, idx_ref[...])
        X = xyzT_ref[0]
        Y = xyzT_ref[1]
        Z = xyzT_ref[2]
        mask = col == farthest
        cx = jnp.sum(jnp.where(mask, X, 0.0), axis=1, keepdims=True)
        cy = jnp.sum(jnp.where(mask, Y, 0.0), axis=1, keepdims=True)
        cz = jnp.sum(jnp.where(mask, Z, 0.0), axis=1, keepdims=True)
        dx = X - cx
        dy = Y - cy
        dz = Z - cz
        dist = dx * dx + dy * dy + dz * dz
        distance = jnp.minimum(dist_scratch[...], dist)
        dist_scratch[...] = distance
        m = jnp.max(distance, axis=1, keepdims=True)
        far_scratch[...] = jnp.min(
            jnp.where(distance == m, col, N), axis=1, keepdims=True
        ).astype(jnp.int32)
        return 0

    lax.fori_loop(0, _G, body, 0)


def _fps(xyzT):
    B, N = xyzT.shape[1], xyzT.shape[2]
    return pl.pallas_call(
        _fps_body,
        out_shape=jax.ShapeDtypeStruct((B, _G), jnp.int32),
        scratch_shapes=[
            pltpu.VMEM((B, N), jnp.float32),
            pltpu.VMEM((B, 1), jnp.int32),
        ],
    )(xyzT)


_Q = 8


def _knn_body(xyzBT_ref, idx_ref, knn_ref, qT_ref):
    N = xyzBT_ref.shape[2]
    col = lax.broadcasted_iota(jnp.int32, (_Q, N), 1)
    kcol = lax.broadcasted_iota(jnp.int32, (_Q, _K), 1)
    qidx = idx_ref[0, 0]
    X = jnp.broadcast_to(xyzBT_ref[0, 0].reshape(1, N), (_Q, N))
    Y = jnp.broadcast_to(xyzBT_ref[0, 1].reshape(1, N), (_Q, N))
    Z = jnp.broadcast_to(xyzBT_ref[0, 2].reshape(1, N), (_Q, N))
    mask = col == qidx
    qx = jnp.sum(jnp.where(mask, X, 0.0), axis=1, keepdims=True)
    qy = jnp.sum(jnp.where(mask, Y, 0.0), axis=1, keepdims=True)
    qz = jnp.sum(jnp.where(mask, Z, 0.0), axis=1, keepdims=True)
    qT_ref[0, 0, 0] = qx
    qT_ref[1, 0, 0] = qy
    qT_ref[2, 0, 0] = qz
    q = jnp.concatenate([qx, qy, qz], axis=1)
    dst = xyzBT_ref[0]
    mm = lax.dot_general(q, dst, (((1,), (0,)), ((), ())),
                         preferred_element_type=jnp.float32)
    nq = jnp.sum(q * q, axis=1, keepdims=True)
    nd = jnp.sum(dst * dst, axis=0, keepdims=True)
    D0 = -2.0 * mm
    D0 = D0 + nq
    D0 = D0 + nd

    knn_ref[0, 0] = jnp.zeros((_Q, _K), jnp.int32)

    def body(k, D):
        m = jnp.min(D, axis=1, keepdims=True)
        sel = D == m
        idxk = jnp.min(jnp.where(sel, col, N), axis=1, keepdims=True)
        knn_ref[0, 0] = jnp.where(kcol == k, idxk, knn_ref[0, 0])
        return jnp.where(col == idxk, _INF, D)

    lax.fori_loop(0, _K, body, D0)


def _knn(xyzBT, idx4):
    B, N = xyzBT.shape[0], xyzBT.shape[2]
    QB = _G // _Q
    grid = (B, QB)
    return pl.pallas_call(
        _knn_body,
        grid=grid,
        in_specs=[
            pl.BlockSpec((1, 3, N), lambda b, q: (b, 0, 0)),
            pl.BlockSpec((1, 1, _Q, 1), lambda b, q: (b, q, 0, 0)),
        ],
        out_specs=[
            pl.BlockSpec((1, 1, _Q, _K), lambda b, q: (b, q, 0, 0)),
            pl.BlockSpec((3, 1, 1, _Q, 1), lambda b, q: (0, b, q, 0, 0)),
        ],
        out_shape=[
            jax.ShapeDtypeStruct((B, QB, _Q, _K), jnp.int32),
            jax.ShapeDtypeStruct((3, B, QB, _Q, 1), jnp.float32),
        ],
        compiler_params=pltpu.CompilerParams(
            dimension_semantics=("arbitrary", "arbitrary"),
        ),
    )(xyzBT, idx4)


def kernel(xyz):
    B, N, C = xyz.shape
    xyzT = jnp.transpose(xyz, (2, 0, 1))
    idx = _fps(xyzT)
    idx4 = idx.reshape(B, _G // _Q, _Q, 1)
    xyzBT = jnp.transpose(xyz, (0, 2, 1))
    knn4, qT5 = _knn(xyzBT, idx4)
    knn_idx = knn4.reshape(B, _G, _K)
    new_xyz = jnp.transpose(qT5.reshape(3, B, _G), (1, 2, 0))

    bidx = jnp.arange(B).reshape(B, 1, 1)
    grouped = xyz[bidx, knn_idx, :]
    d = grouped - new_xyz[:, :, None, :]
    std = jnp.std(d, ddof=1)
    knn_xyz = d / (std + 1e-5)
    tiled = jnp.broadcast_to(
        new_xyz.reshape(B, _G, 1, C), (B, _G, _K, C)
    )
    knn_xyz = jnp.concatenate([knn_xyz, tiled], axis=-1)
    return (new_xyz, knn_xyz)


# speedup vs baseline: 2.7195x; 1.6618x over previous
import functools

import jax
import jax.numpy as jnp
from jax import lax
from jax.experimental import pallas as pl
from jax.experimental.pallas import tpu as pltpu

_G = 1024
_K = 32
_INF = 3.0e38


def _fps_body(xyzT_ref, idx_ref, dist_scratch, far_scratch):
    B = xyzT_ref.shape[1]
    N = xyzT_ref.shape[2]
    col = lax.broadcasted_iota(jnp.int32, (B, N), 1)
    gcol = lax.broadcasted_iota(jnp.int32, (B, _G), 1)
    dist_scratch[...] = jnp.full((B, N), 1e10, jnp.float32)
    far_scratch[...] = jnp.zeros((B, 1), jnp.int32)
    idx_ref[...] = jnp.zeros((B, _G), jnp.int32)

    def body(i, _):
        farthest = far_scratch[...]
        idx_ref[...] = jnp.where(gcol == i, farthest, idx_ref[...])
        X = xyzT_ref[0]
        Y = xyzT_ref[1]
        Z = xyzT_ref[2]
        mask = col == farthest
        cx = jnp.sum(jnp.where(mask, X, 0.0), axis=1, keepdims=True)
        cy = jnp.sum(jnp.where(mask, Y, 0.0), axis=1, keepdims=True)
        cz = jnp.sum(jnp.where(mask, Z, 0.0), axis=1, keepdims=True)
        dx = X - cx
        dy = Y - cy
        dz = Z - cz
        dist = dx * dx + dy * dy + dz * dz
        distance = jnp.minimum(dist_scratch[...], dist)
        dist_scratch[...] = distance
        m = jnp.max(distance, axis=1, keepdims=True)
        far_scratch[...] = jnp.min(
            jnp.where(distance == m, col, N), axis=1, keepdims=True
        ).astype(jnp.int32)
        return 0

    lax.fori_loop(0, _G, body, 0)


def _fps(xyzT):
    B, N = xyzT.shape[1], xyzT.shape[2]
    return pl.pallas_call(
        _fps_body,
        out_shape=jax.ShapeDtypeStruct((B, _G), jnp.int32),
        scratch_shapes=[
            pltpu.VMEM((B, N), jnp.float32),
            pltpu.VMEM((B, 1), jnp.int32),
        ],
    )(xyzT)


_Q = 32


def _knn_body(xyzBT_ref, idx_ref, knn_ref, qT_ref):
    N = xyzBT_ref.shape[2]
    col = lax.broadcasted_iota(jnp.int32, (_Q, N), 1)
    kcol = lax.broadcasted_iota(jnp.int32, (_Q, _K), 1)
    qidx = idx_ref[0, 0]
    X = jnp.broadcast_to(xyzBT_ref[0, 0].reshape(1, N), (_Q, N))
    Y = jnp.broadcast_to(xyzBT_ref[0, 1].reshape(1, N), (_Q, N))
    Z = jnp.broadcast_to(xyzBT_ref[0, 2].reshape(1, N), (_Q, N))
    mask = col == qidx
    qx = jnp.sum(jnp.where(mask, X, 0.0), axis=1, keepdims=True)
    qy = jnp.sum(jnp.where(mask, Y, 0.0), axis=1, keepdims=True)
    qz = jnp.sum(jnp.where(mask, Z, 0.0), axis=1, keepdims=True)
    qT_ref[0, 0, 0] = qx
    qT_ref[1, 0, 0] = qy
    qT_ref[2, 0, 0] = qz
    q = jnp.concatenate([qx, qy, qz], axis=1)
    dst = xyzBT_ref[0]
    mm = lax.dot_general(q, dst, (((1,), (0,)), ((), ())),
                         preferred_element_type=jnp.float32)
    nq = jnp.sum(q * q, axis=1, keepdims=True)
    nd = jnp.sum(dst * dst, axis=0, keepdims=True)
    D0 = -2.0 * mm
    D0 = D0 + nq
    D0 = D0 + nd

    knn_ref[0, 0] = jnp.zeros((_Q, _K), jnp.int32)

    def body(k, D):
        m = jnp.min(D, axis=1, keepdims=True)
        sel = D == m
        idxk = jnp.min(jnp.where(sel, col, N), axis=1, keepdims=True)
        knn_ref[0, 0] = jnp.where(kcol == k, idxk, knn_ref[0, 0])
        return jnp.where(col == idxk, _INF, D)

    lax.fori_loop(0, _K, body, D0)


def _knn(xyzBT, idx4):
    B, N = xyzBT.shape[0], xyzBT.shape[2]
    QB = _G // _Q
    grid = (B, QB)
    return pl.pallas_call(
        _knn_body,
        grid=grid,
        in_specs=[
            pl.BlockSpec((1, 3, N), lambda b, q: (b, 0, 0)),
            pl.BlockSpec((1, 1, _Q, 1), lambda b, q: (b, q, 0, 0)),
        ],
        out_specs=[
            pl.BlockSpec((1, 1, _Q, _K), lambda b, q: (b, q, 0, 0)),
            pl.BlockSpec((3, 1, 1, _Q, 1), lambda b, q: (0, b, q, 0, 0)),
        ],
        out_shape=[
            jax.ShapeDtypeStruct((B, QB, _Q, _K), jnp.int32),
            jax.ShapeDtypeStruct((3, B, QB, _Q, 1), jnp.float32),
        ],
        compiler_params=pltpu.CompilerParams(
            dimension_semantics=("arbitrary", "arbitrary"),
        ),
    )(xyzBT, idx4)


def kernel(xyz):
    B, N, C = xyz.shape
    xyzT = jnp.transpose(xyz, (2, 0, 1))
    idx = _fps(xyzT)
    idx4 = idx.reshape(B, _G // _Q, _Q, 1)
    xyzBT = jnp.transpose(xyz, (0, 2, 1))
    knn4, qT5 = _knn(xyzBT, idx4)
    knn_idx = knn4.reshape(B, _G, _K)
    new_xyz = jnp.transpose(qT5.reshape(3, B, _G), (1, 2, 0))

    bidx = jnp.arange(B).reshape(B, 1, 1)
    grouped = xyz[bidx, knn_idx, :]
    d = grouped - new_xyz[:, :, None, :]
    std = jnp.std(d, ddof=1)
    knn_xyz = d / (std + 1e-5)
    tiled = jnp.broadcast_to(
        new_xyz.reshape(B, _G, 1, C), (B, _G, _K, C)
    )
    knn_xyz = jnp.concatenate([knn_xyz, tiled], axis=-1)
    return (new_xyz, knn_xyz)


# speedup vs baseline: 5.3240x; 1.9577x over previous
import functools

import jax
import jax.numpy as jnp
from jax import lax
from jax.experimental import pallas as pl
from jax.experimental.pallas import tpu as pltpu
from jax.experimental.pallas import tpu_sc as plsc

_G = 1024
_K = 32
_INF = 3.0e38


def _fps_body(xyzT_ref, idx_ref, dist_scratch, far_scratch):
    B = xyzT_ref.shape[1]
    N = xyzT_ref.shape[2]
    col = lax.broadcasted_iota(jnp.int32, (B, N), 1)
    gcol = lax.broadcasted_iota(jnp.int32, (B, _G), 1)
    dist_scratch[...] = jnp.full((B, N), 1e10, jnp.float32)
    far_scratch[...] = jnp.zeros((B, 1), jnp.int32)
    idx_ref[...] = jnp.zeros((B, _G), jnp.int32)

    def body(i, _):
        farthest = far_scratch[...]
        idx_ref[...] = jnp.where(gcol == i, farthest, idx_ref[...])
        X = xyzT_ref[0]
        Y = xyzT_ref[1]
        Z = xyzT_ref[2]
        mask = col == farthest
        cx = jnp.sum(jnp.where(mask, X, 0.0), axis=1, keepdims=True)
        cy = jnp.sum(jnp.where(mask, Y, 0.0), axis=1, keepdims=True)
        cz = jnp.sum(jnp.where(mask, Z, 0.0), axis=1, keepdims=True)
        dx = X - cx
        dy = Y - cy
        dz = Z - cz
        dist = dx * dx + dy * dy + dz * dz
        distance = jnp.minimum(dist_scratch[...], dist)
        dist_scratch[...] = distance
        m = jnp.max(distance, axis=1, keepdims=True)
        far_scratch[...] = jnp.min(
            jnp.where(distance == m, col, N), axis=1, keepdims=True
        ).astype(jnp.int32)
        return 0

    lax.fori_loop(0, _G, body, 0)


def _fps(xyzT):
    B, N = xyzT.shape[1], xyzT.shape[2]
    return pl.pallas_call(
        _fps_body,
        out_shape=jax.ShapeDtypeStruct((B, _G), jnp.int32),
        scratch_shapes=[
            pltpu.VMEM((B, N), jnp.float32),
            pltpu.VMEM((B, 1), jnp.int32),
        ],
    )(xyzT)


_Q = 32


def _knn_body(xyzBT_ref, idx_ref, knn_ref, qT_ref):
    N = xyzBT_ref.shape[2]
    col = lax.broadcasted_iota(jnp.int32, (_Q, N), 1)
    kcol = lax.broadcasted_iota(jnp.int32, (_Q, _K), 1)
    qidx = idx_ref[0, 0]
    X = jnp.broadcast_to(xyzBT_ref[0, 0].reshape(1, N), (_Q, N))
    Y = jnp.broadcast_to(xyzBT_ref[0, 1].reshape(1, N), (_Q, N))
    Z = jnp.broadcast_to(xyzBT_ref[0, 2].reshape(1, N), (_Q, N))
    mask = col == qidx
    qx = jnp.sum(jnp.where(mask, X, 0.0), axis=1, keepdims=True)
    qy = jnp.sum(jnp.where(mask, Y, 0.0), axis=1, keepdims=True)
    qz = jnp.sum(jnp.where(mask, Z, 0.0), axis=1, keepdims=True)
    qT_ref[0, 0, 0] = qx
    qT_ref[1, 0, 0] = qy
    qT_ref[2, 0, 0] = qz
    q = jnp.concatenate([qx, qy, qz], axis=1)
    dst = xyzBT_ref[0]
    mm = lax.dot_general(q, dst, (((1,), (0,)), ((), ())),
                         preferred_element_type=jnp.float32)
    nq = jnp.sum(q * q, axis=1, keepdims=True)
    nd = jnp.sum(dst * dst, axis=0, keepdims=True)
    D0 = -2.0 * mm
    D0 = D0 + nq
    D0 = D0 + nd

    knn_ref[0, 0] = jnp.zeros((_Q, _K), jnp.int32)

    def body(k, D):
        m = jnp.min(D, axis=1, keepdims=True)
        sel = D == m
        idxk = jnp.min(jnp.where(sel, col, N), axis=1, keepdims=True)
        knn_ref[0, 0] = jnp.where(kcol == k, idxk, knn_ref[0, 0])
        return jnp.where(col == idxk, _INF, D)

    lax.fori_loop(0, _K, body, D0)
    knn_ref[0, 0] = knn_ref[0, 0] + pl.program_id(0) * N


def _knn(xyzBT, idx4):
    B, N = xyzBT.shape[0], xyzBT.shape[2]
    QB = _G // _Q
    grid = (B, QB)
    return pl.pallas_call(
        _knn_body,
        grid=grid,
        in_specs=[
            pl.BlockSpec((1, 3, N), lambda b, q: (b, 0, 0)),
            pl.BlockSpec((1, 1, _Q, 1), lambda b, q: (b, q, 0, 0)),
        ],
        out_specs=[
            pl.BlockSpec((1, 1, _Q, _K), lambda b, q: (b, q, 0, 0)),
            pl.BlockSpec((3, 1, 1, _Q, 1), lambda b, q: (0, b, q, 0, 0)),
        ],
        out_shape=[
            jax.ShapeDtypeStruct((B, QB, _Q, _K), jnp.int32),
            jax.ShapeDtypeStruct((3, B, QB, _Q, 1), jnp.float32),
        ],
        compiler_params=pltpu.CompilerParams(
            dimension_semantics=("arbitrary", "arbitrary"),
        ),
    )(xyzBT, idx4)


_NC, _NS = 2, 16
_NW = _NC * _NS


def _sc_gather(table, idx2):
    BR = idx2.shape[0] * 128
    rows_per_w = BR // _NW
    n_chunks = rows_per_w // 128
    mesh = plsc.VectorSubcoreMesh(core_axis_name="c", subcore_axis_name="s")

    @functools.partial(
        pl.kernel,
        mesh=mesh,
        out_type=jax.ShapeDtypeStruct((BR, 16), jnp.float32),
        compiler_params=pltpu.CompilerParams(use_tc_tiling_on_sc=False),
        scratch_types=[
            pltpu.VMEM((n_chunks, 128), jnp.int32),
            pltpu.VMEM((128, 16), jnp.float32),
            pltpu.VMEM((128, 16), jnp.float32),
            pltpu.SemaphoreType.DMA,
            pltpu.SemaphoreType.DMA,
        ],
    )
    def k(table_hbm, idx_hbm, out_hbm, idx_v, buf0, buf1, sem0, sem1):
        wid = lax.axis_index("s") * _NC + lax.axis_index("c")
        base = wid * rows_per_w
        pltpu.sync_copy(idx_hbm.at[pl.ds(wid * n_chunks, n_chunks)], idx_v)
        bufs = (buf0, buf1)
        sems = (sem0, sem1)
        handles = [
            pltpu.async_copy(table_hbm.at[idx_v.at[0]], buf0, sem0),
            pltpu.async_copy(table_hbm.at[idx_v.at[1]], buf1, sem1),
        ]
        for c in range(n_chunks):
            b = c % 2
            handles[b].wait()
            pltpu.sync_copy(bufs[b], out_hbm.at[pl.ds(base + c * 128, 128)])
            if c + 2 < n_chunks:
                handles[b] = pltpu.async_copy(
                    table_hbm.at[idx_v.at[c + 2]], bufs[b], sems[b]
                )

    return k(table, idx2)


_NBLK = 32


def _sums_body(g_ref, q_ref, s1_ref, s2_ref):
    d = g_ref[...] - q_ref[...]
    s1_ref[0] = jnp.sum(d, axis=0, keepdims=True)
    s2_ref[0] = jnp.sum(d * d, axis=0, keepdims=True)


def _sums(g2, q2):
    R = g2.shape[0]
    blk = R // _NBLK
    return pl.pallas_call(
        _sums_body,
        grid=(_NBLK,),
        in_specs=[
            pl.BlockSpec((blk, 128), lambda c: (c, 0)),
            pl.BlockSpec((blk, 128), lambda c: (c, 0)),
        ],
        out_specs=[
            pl.BlockSpec((1, 1, 128), lambda c: (c, 0, 0)),
            pl.BlockSpec((1, 1, 128), lambda c: (c, 0, 0)),
        ],
        out_shape=[
            jax.ShapeDtypeStruct((_NBLK, 1, 128), jnp.float32),
            jax.ShapeDtypeStruct((_NBLK, 1, 128), jnp.float32),
        ],
    )(g2, q2)


_M = 8 * _G * _K * 3


def _scale_body(g_ref, q_ref, s1_ref, s2_ref, out_ref):
    S1 = jnp.sum(s1_ref[...])
    S2 = jnp.sum(s2_ref[...])
    var = (S2 - S1 * S1 / _M) / (_M - 1)
    std = jnp.sqrt(var)
    out_ref[...] = (g_ref[...] - q_ref[...]) / (std + 1e-5)


def _scale(g2, q2, s1, s2):
    R = g2.shape[0]
    blk = R // _NBLK
    return pl.pallas_call(
        _scale_body,
        grid=(_NBLK,),
        in_specs=[
            pl.BlockSpec((blk, 128), lambda c: (c, 0)),
            pl.BlockSpec((blk, 128), lambda c: (c, 0)),
            pl.BlockSpec((_NBLK, 1, 128), lambda c: (0, 0, 0)),
            pl.BlockSpec((_NBLK, 1, 128), lambda c: (0, 0, 0)),
        ],
        out_specs=pl.BlockSpec((blk, 128), lambda c: (c, 0)),
        out_shape=jax.ShapeDtypeStruct((R, 128), jnp.float32),
    )(g2, q2, s1, s2)


def kernel(xyz):
    B, N, C = xyz.shape
    xyzT = jnp.transpose(xyz, (2, 0, 1))
    idx = _fps(xyzT)
    idx4 = idx.reshape(B, _G // _Q, _Q, 1)
    xyzBT = jnp.transpose(xyz, (0, 2, 1))
    knn4, qT5 = _knn(xyzBT, idx4)
    new_xyz = jnp.transpose(qT5.reshape(3, B, _G), (1, 2, 0))

    BR = B * _G * _K
    table = jnp.pad(xyz.reshape(B * N, C), ((0, 0), (0, 16 - C)))
    idx2 = knn4.reshape(BR // 128, 128)
    g = _sc_gather(table, idx2)

    q16 = jnp.pad(new_xyz.reshape(B * _G, C), ((0, 0), (0, 16 - C)))
    R = BR * 16 // 128
    g2 = g.reshape(R, 128)
    q2 = jnp.broadcast_to(
        q16[:, None, None, :], (B * _G, 4, 8, 16)
    ).reshape(R, 128)
    s1, s2 = _sums(g2, q2)
    d2 = _scale(g2, q2, s1, s2)

    knn_xyz = d2.reshape(BR, 16)[:, :C].reshape(B, _G, _K, C)
    tiled = jnp.broadcast_to(
        new_xyz.reshape(B, _G, 1, C), (B, _G, _K, C)
    )
    knn_xyz = jnp.concatenate([knn_xyz, tiled], axis=-1)
    return (new_xyz, knn_xyz)


# speedup vs baseline: 7.2161x; 1.3554x over previous
import functools

import jax
import jax.numpy as jnp
from jax import lax
from jax.experimental import pallas as pl
from jax.experimental.pallas import tpu as pltpu
from jax.experimental.pallas import tpu_sc as plsc

_G = 1024
_K = 32
_INF = 3.0e38


def _fps_body(xyzT_ref, idx_ref, dist_scratch, far_scratch):
    B = xyzT_ref.shape[1]
    N = xyzT_ref.shape[2]
    col = lax.broadcasted_iota(jnp.int32, (B, N), 1)
    gcol = lax.broadcasted_iota(jnp.int32, (B, _G), 1)
    dist_scratch[...] = jnp.full((B, N), 1e10, jnp.float32)
    far_scratch[...] = jnp.zeros((B, 1), jnp.int32)
    idx_ref[...] = jnp.zeros((B, _G), jnp.int32)

    def body(i, _):
        farthest = far_scratch[...]
        idx_ref[...] = jnp.where(gcol == i, farthest, idx_ref[...])
        X = xyzT_ref[0]
        Y = xyzT_ref[1]
        Z = xyzT_ref[2]
        mask = col == farthest
        cx = jnp.sum(jnp.where(mask, X, 0.0), axis=1, keepdims=True)
        cy = jnp.sum(jnp.where(mask, Y, 0.0), axis=1, keepdims=True)
        cz = jnp.sum(jnp.where(mask, Z, 0.0), axis=1, keepdims=True)
        dx = X - cx
        dy = Y - cy
        dz = Z - cz
        dist = dx * dx + dy * dy + dz * dz
        distance = jnp.minimum(dist_scratch[...], dist)
        dist_scratch[...] = distance
        m = jnp.max(distance, axis=1, keepdims=True)
        far_scratch[...] = jnp.min(
            jnp.where(distance == m, col, N), axis=1, keepdims=True
        ).astype(jnp.int32)
        return 0

    lax.fori_loop(0, _G, body, 0)


def _fps(xyzT):
    B, N = xyzT.shape[1], xyzT.shape[2]
    return pl.pallas_call(
        _fps_body,
        out_shape=jax.ShapeDtypeStruct((B, _G), jnp.int32),
        scratch_shapes=[
            pltpu.VMEM((B, N), jnp.float32),
            pltpu.VMEM((B, 1), jnp.int32),
        ],
    )(xyzT)


_Q = 64


def _knn_body(xyzBT_ref, idx_ref, knn_ref, qT_ref):
    N = xyzBT_ref.shape[2]
    col = lax.broadcasted_iota(jnp.int32, (_Q, N), 1)
    kcol = lax.broadcasted_iota(jnp.int32, (_Q, _K), 1)
    qidx = idx_ref[0, 0]
    X = jnp.broadcast_to(xyzBT_ref[0, 0].reshape(1, N), (_Q, N))
    Y = jnp.broadcast_to(xyzBT_ref[0, 1].reshape(1, N), (_Q, N))
    Z = jnp.broadcast_to(xyzBT_ref[0, 2].reshape(1, N), (_Q, N))
    mask = col == qidx
    qx = jnp.sum(jnp.where(mask, X, 0.0), axis=1, keepdims=True)
    qy = jnp.sum(jnp.where(mask, Y, 0.0), axis=1, keepdims=True)
    qz = jnp.sum(jnp.where(mask, Z, 0.0), axis=1, keepdims=True)
    qT_ref[0, 0, 0] = qx
    qT_ref[1, 0, 0] = qy
    qT_ref[2, 0, 0] = qz
    q = jnp.concatenate([qx, qy, qz], axis=1)
    dst = xyzBT_ref[0]
    mm = lax.dot_general(q, dst, (((1,), (0,)), ((), ())),
                         preferred_element_type=jnp.float32)
    nq = jnp.sum(q * q, axis=1, keepdims=True)
    nd = jnp.sum(dst * dst, axis=0, keepdims=True)
    D0 = -2.0 * mm
    D0 = D0 + nq
    D0 = D0 + nd

    knn_ref[0, 0] = jnp.zeros((_Q, _K), jnp.int32)

    def body(k, D):
        m = jnp.min(D, axis=1, keepdims=True)
        sel = D == m
        idxk = jnp.min(jnp.where(sel, col, N), axis=1, keepdims=True)
        knn_ref[0, 0] = jnp.where(kcol == k, idxk, knn_ref[0, 0])
        return jnp.where(col == idxk, _INF, D)

    lax.fori_loop(0, _K, body, D0)
    knn_ref[0, 0] = knn_ref[0, 0] + pl.program_id(0) * N


def _knn(xyzBT, idx4):
    B, N = xyzBT.shape[0], xyzBT.shape[2]
    QB = _G // _Q
    grid = (B, QB)
    return pl.pallas_call(
        _knn_body,
        grid=grid,
        in_specs=[
            pl.BlockSpec((1, 3, N), lambda b, q: (b, 0, 0)),
            pl.BlockSpec((1, 1, _Q, 1), lambda b, q: (b, q, 0, 0)),
        ],
        out_specs=[
            pl.BlockSpec((1, 1, _Q, _K), lambda b, q: (b, q, 0, 0)),
            pl.BlockSpec((3, 1, 1, _Q, 1), lambda b, q: (0, b, q, 0, 0)),
        ],
        out_shape=[
            jax.ShapeDtypeStruct((B, QB, _Q, _K), jnp.int32),
            jax.ShapeDtypeStruct((3, B, QB, _Q, 1), jnp.float32),
        ],
        compiler_params=pltpu.CompilerParams(
            dimension_semantics=("arbitrary", "arbitrary"),
        ),
    )(xyzBT, idx4)


_NC, _NS = 2, 16
_NW = _NC * _NS


def _sc_gather(table, idx2):
    BR = idx2.shape[0] * 128
    rows_per_w = BR // _NW
    n_chunks = rows_per_w // 128
    mesh = plsc.VectorSubcoreMesh(core_axis_name="c", subcore_axis_name="s")

    @functools.partial(
        pl.kernel,
        mesh=mesh,
        out_type=jax.ShapeDtypeStruct((BR, 16), jnp.float32),
        compiler_params=pltpu.CompilerParams(use_tc_tiling_on_sc=False),
        scratch_types=[
            pltpu.VMEM((n_chunks, 128), jnp.int32),
            pltpu.VMEM((128, 16), jnp.float32),
            pltpu.VMEM((128, 16), jnp.float32),
            pltpu.SemaphoreType.DMA,
            pltpu.SemaphoreType.DMA,
        ],
    )
    def k(table_hbm, idx_hbm, out_hbm, idx_v, buf0, buf1, sem0, sem1):
        wid = lax.axis_index("s") * _NC + lax.axis_index("c")
        base = wid * rows_per_w
        pltpu.sync_copy(idx_hbm.at[pl.ds(wid * n_chunks, n_chunks)], idx_v)
        bufs = (buf0, buf1)
        sems = (sem0, sem1)
        handles = [
            pltpu.async_copy(table_hbm.at[idx_v.at[0]], buf0, sem0),
            pltpu.async_copy(table_hbm.at[idx_v.at[1]], buf1, sem1),
        ]
        for c in range(n_chunks):
            b = c % 2
            handles[b].wait()
            pltpu.sync_copy(bufs[b], out_hbm.at[pl.ds(base + c * 128, 128)])
            if c + 2 < n_chunks:
                handles[b] = pltpu.async_copy(
                    table_hbm.at[idx_v.at[c + 2]], bufs[b], sems[b]
                )

    return k(table, idx2)


_NBLK = 32


def _sums_body(g_ref, q_ref, s1_ref, s2_ref):
    d = g_ref[...] - q_ref[...]
    s1_ref[0] = jnp.sum(d, axis=0, keepdims=True)
    s2_ref[0] = jnp.sum(d * d, axis=0, keepdims=True)


def _sums(g2, q2):
    R = g2.shape[0]
    blk = R // _NBLK
    return pl.pallas_call(
        _sums_body,
        grid=(_NBLK,),
        in_specs=[
            pl.BlockSpec((blk, 128), lambda c: (c, 0)),
            pl.BlockSpec((blk, 128), lambda c: (c, 0)),
        ],
        out_specs=[
            pl.BlockSpec((1, 1, 128), lambda c: (c, 0, 0)),
            pl.BlockSpec((1, 1, 128), lambda c: (c, 0, 0)),
        ],
        out_shape=[
            jax.ShapeDtypeStruct((_NBLK, 1, 128), jnp.float32),
            jax.ShapeDtypeStruct((_NBLK, 1, 128), jnp.float32),
        ],
    )(g2, q2)


_M = 8 * _G * _K * 3


def _scale_body(g_ref, q_ref, s1_ref, s2_ref, out_ref):
    S1 = jnp.sum(s1_ref[...])
    S2 = jnp.sum(s2_ref[...])
    var = (S2 - S1 * S1 / _M) / (_M - 1)
    std = jnp.sqrt(var)
    out_ref[...] = (g_ref[...] - q_ref[...]) / (std + 1e-5)


def _scale(g2, q2, s1, s2):
    R = g2.shape[0]
    blk = R // _NBLK
    return pl.pallas_call(
        _scale_body,
        grid=(_NBLK,),
        in_specs=[
            pl.BlockSpec((blk, 128), lambda c: (c, 0)),
            pl.BlockSpec((blk, 128), lambda c: (c, 0)),
            pl.BlockSpec((_NBLK, 1, 128), lambda c: (0, 0, 0)),
            pl.BlockSpec((_NBLK, 1, 128), lambda c: (0, 0, 0)),
        ],
        out_specs=pl.BlockSpec((blk, 128), lambda c: (c, 0)),
        out_shape=jax.ShapeDtypeStruct((R, 128), jnp.float32),
    )(g2, q2, s1, s2)


def kernel(xyz):
    B, N, C = xyz.shape
    xyzT = jnp.transpose(xyz, (2, 0, 1))
    idx = _fps(xyzT)
    idx4 = idx.reshape(B, _G // _Q, _Q, 1)
    xyzBT = jnp.transpose(xyz, (0, 2, 1))
    knn4, qT5 = _knn(xyzBT, idx4)
    new_xyz = jnp.transpose(qT5.reshape(3, B, _G), (1, 2, 0))

    BR = B * _G * _K
    table = jnp.pad(xyz.reshape(B * N, C), ((0, 0), (0, 16 - C)))
    idx2 = knn4.reshape(BR // 128, 128)
    g = _sc_gather(table, idx2)

    q16 = jnp.pad(new_xyz.reshape(B * _G, C), ((0, 0), (0, 16 - C)))
    R = BR * 16 // 128
    g2 = g.reshape(R, 128)
    q2 = jnp.broadcast_to(
        q16[:, None, None, :], (B * _G, 4, 8, 16)
    ).reshape(R, 128)
    s1, s2 = _sums(g2, q2)
    d2 = _scale(g2, q2, s1, s2)

    knn_xyz = d2.reshape(BR, 16)[:, :C].reshape(B, _G, _K, C)
    tiled = jnp.broadcast_to(
        new_xyz.reshape(B, _G, 1, C), (B, _G, _K, C)
    )
    knn_xyz = jnp.concatenate([knn_xyz, tiled], axis=-1)
    return (new_xyz, knn_xyz)


# speedup vs baseline: 7.7802x; 1.0782x over previous
import functools

import jax
import jax.numpy as jnp
from jax import lax
from jax.experimental import pallas as pl
from jax.experimental.pallas import tpu as pltpu
from jax.experimental.pallas import tpu_sc as plsc

_G = 1024
_K = 32
_INF = 3.0e38


def _fps_body(xyzT_ref, idx_ref, dist_scratch, far_scratch):
    B = xyzT_ref.shape[1]
    N = xyzT_ref.shape[2]
    col = lax.broadcasted_iota(jnp.int32, (B, N), 1)
    gcol = lax.broadcasted_iota(jnp.int32, (B, _G), 1)
    dist_scratch[...] = jnp.full((B, N), 1e10, jnp.float32)
    far_scratch[...] = jnp.zeros((B, 1), jnp.int32)
    idx_ref[...] = jnp.zeros((B, _G), jnp.int32)

    def body(i, _):
        farthest = far_scratch[...]
        idx_ref[...] = jnp.where(gcol == i, farthest, idx_ref[...])
        X = xyzT_ref[0]
        Y = xyzT_ref[1]
        Z = xyzT_ref[2]
        mask = col == farthest
        cx = jnp.sum(jnp.where(mask, X, 0.0), axis=1, keepdims=True)
        cy = jnp.sum(jnp.where(mask, Y, 0.0), axis=1, keepdims=True)
        cz = jnp.sum(jnp.where(mask, Z, 0.0), axis=1, keepdims=True)
        dx = X - cx
        dy = Y - cy
        dz = Z - cz
        dist = dx * dx + dy * dy + dz * dz
        distance = jnp.minimum(dist_scratch[...], dist)
        dist_scratch[...] = distance
        m = jnp.max(distance, axis=1, keepdims=True)
        far_scratch[...] = jnp.min(
            jnp.where(distance == m, col, N), axis=1, keepdims=True
        ).astype(jnp.int32)
        return 0

    lax.fori_loop(0, _G, body, 0)


def _fps(xyzT):
    B, N = xyzT.shape[1], xyzT.shape[2]
    return pl.pallas_call(
        _fps_body,
        out_shape=jax.ShapeDtypeStruct((B, _G), jnp.int32),
        scratch_shapes=[
            pltpu.VMEM((B, N), jnp.float32),
            pltpu.VMEM((B, 1), jnp.int32),
        ],
    )(xyzT)


_Q = 128


def _knn_body(xyzBT_ref, idx_ref, knn_ref, qT_ref):
    N = xyzBT_ref.shape[2]
    col = lax.broadcasted_iota(jnp.int32, (_Q, N), 1)
    kcol = lax.broadcasted_iota(jnp.int32, (_Q, _K), 1)
    qidx = idx_ref[0, 0]
    X = jnp.broadcast_to(xyzBT_ref[0, 0].reshape(1, N), (_Q, N))
    Y = jnp.broadcast_to(xyzBT_ref[0, 1].reshape(1, N), (_Q, N))
    Z = jnp.broadcast_to(xyzBT_ref[0, 2].reshape(1, N), (_Q, N))
    mask = col == qidx
    qx = jnp.sum(jnp.where(mask, X, 0.0), axis=1, keepdims=True)
    qy = jnp.sum(jnp.where(mask, Y, 0.0), axis=1, keepdims=True)
    qz = jnp.sum(jnp.where(mask, Z, 0.0), axis=1, keepdims=True)
    qT_ref[0, 0, 0] = qx
    qT_ref[1, 0, 0] = qy
    qT_ref[2, 0, 0] = qz
    q = jnp.concatenate([qx, qy, qz], axis=1)
    dst = xyzBT_ref[0]
    mm = lax.dot_general(q, dst, (((1,), (0,)), ((), ())),
                         preferred_element_type=jnp.float32)
    nq = jnp.sum(q * q, axis=1, keepdims=True)
    nd = jnp.sum(dst * dst, axis=0, keepdims=True)
    D0 = -2.0 * mm
    D0 = D0 + nq
    D0 = D0 + nd

    knn_ref[0, 0] = jnp.zeros((_Q, _K), jnp.int32)

    def body(k, D):
        m = jnp.min(D, axis=1, keepdims=True)
        sel = D == m
        idxk = jnp.min(jnp.where(sel, col, N), axis=1, keepdims=True)
        knn_ref[0, 0] = jnp.where(kcol == k, idxk, knn_ref[0, 0])
        return jnp.where(col == idxk, _INF, D)

    lax.fori_loop(0, _K, body, D0)
    knn_ref[0, 0] = knn_ref[0, 0] + pl.program_id(0) * N


def _knn(xyzBT, idx4):
    B, N = xyzBT.shape[0], xyzBT.shape[2]
    QB = _G // _Q
    grid = (B, QB)
    return pl.pallas_call(
        _knn_body,
        grid=grid,
        in_specs=[
            pl.BlockSpec((1, 3, N), lambda b, q: (b, 0, 0)),
            pl.BlockSpec((1, 1, _Q, 1), lambda b, q: (b, q, 0, 0)),
        ],
        out_specs=[
            pl.BlockSpec((1, 1, _Q, _K), lambda b, q: (b, q, 0, 0)),
            pl.BlockSpec((3, 1, 1, _Q, 1), lambda b, q: (0, b, q, 0, 0)),
        ],
        out_shape=[
            jax.ShapeDtypeStruct((B, QB, _Q, _K), jnp.int32),
            jax.ShapeDtypeStruct((3, B, QB, _Q, 1), jnp.float32),
        ],
        compiler_params=pltpu.CompilerParams(
            dimension_semantics=("arbitrary", "arbitrary"),
        ),
    )(xyzBT, idx4)


_NC, _NS = 2, 16
_NW = _NC * _NS


def _sc_gather(table, idx2):
    BR = idx2.shape[0] * 128
    rows_per_w = BR // _NW
    n_chunks = rows_per_w // 128
    mesh = plsc.VectorSubcoreMesh(core_axis_name="c", subcore_axis_name="s")

    @functools.partial(
        pl.kernel,
        mesh=mesh,
        out_type=jax.ShapeDtypeStruct((BR, 16), jnp.float32),
        compiler_params=pltpu.CompilerParams(use_tc_tiling_on_sc=False),
        scratch_types=[
            pltpu.VMEM((n_chunks, 128), jnp.int32),
            pltpu.VMEM((128, 16), jnp.float32),
            pltpu.VMEM((128, 16), jnp.float32),
            pltpu.SemaphoreType.DMA,
            pltpu.SemaphoreType.DMA,
        ],
    )
    def k(table_hbm, idx_hbm, out_hbm, idx_v, buf0, buf1, sem0, sem1):
        wid = lax.axis_index("s") * _NC + lax.axis_index("c")
        base = wid * rows_per_w
        pltpu.sync_copy(idx_hbm.at[pl.ds(wid * n_chunks, n_chunks)], idx_v)
        bufs = (buf0, buf1)
        sems = (sem0, sem1)
        handles = [
            pltpu.async_copy(table_hbm.at[idx_v.at[0]], buf0, sem0),
            pltpu.async_copy(table_hbm.at[idx_v.at[1]], buf1, sem1),
        ]
        for c in range(n_chunks):
            b = c % 2
            handles[b].wait()
            pltpu.sync_copy(bufs[b], out_hbm.at[pl.ds(base + c * 128, 128)])
            if c + 2 < n_chunks:
                handles[b] = pltpu.async_copy(
                    table_hbm.at[idx_v.at[c + 2]], bufs[b], sems[b]
                )

    return k(table, idx2)


_NBLK = 32


def _sums_body(g_ref, q_ref, s1_ref, s2_ref):
    d = g_ref[...] - q_ref[...]
    s1_ref[0] = jnp.sum(d, axis=0, keepdims=True)
    s2_ref[0] = jnp.sum(d * d, axis=0, keepdims=True)


def _sums(g2, q2):
    R = g2.shape[0]
    blk = R // _NBLK
    return pl.pallas_call(
        _sums_body,
        grid=(_NBLK,),
        in_specs=[
            pl.BlockSpec((blk, 128), lambda c: (c, 0)),
            pl.BlockSpec((blk, 128), lambda c: (c, 0)),
        ],
        out_specs=[
            pl.BlockSpec((1, 1, 128), lambda c: (c, 0, 0)),
            pl.BlockSpec((1, 1, 128), lambda c: (c, 0, 0)),
        ],
        out_shape=[
            jax.ShapeDtypeStruct((_NBLK, 1, 128), jnp.float32),
            jax.ShapeDtypeStruct((_NBLK, 1, 128), jnp.float32),
        ],
    )(g2, q2)


_M = 8 * _G * _K * 3


def _scale_body(g_ref, q_ref, s1_ref, s2_ref, out_ref):
    S1 = jnp.sum(s1_ref[...])
    S2 = jnp.sum(s2_ref[...])
    var = (S2 - S1 * S1 / _M) / (_M - 1)
    std = jnp.sqrt(var)
    out_ref[...] = (g_ref[...] - q_ref[...]) / (std + 1e-5)


def _scale(g2, q2, s1, s2):
    R = g2.shape[0]
    blk = R // _NBLK
    return pl.pallas_call(
        _scale_body,
        grid=(_NBLK,),
        in_specs=[
            pl.BlockSpec((blk, 128), lambda c: (c, 0)),
            pl.BlockSpec((blk, 128), lambda c: (c, 0)),
            pl.BlockSpec((_NBLK, 1, 128), lambda c: (0, 0, 0)),
            pl.BlockSpec((_NBLK, 1, 128), lambda c: (0, 0, 0)),
        ],
        out_specs=pl.BlockSpec((blk, 128), lambda c: (c, 0)),
        out_shape=jax.ShapeDtypeStruct((R, 128), jnp.float32),
    )(g2, q2, s1, s2)


def kernel(xyz):
    B, N, C = xyz.shape
    xyzT = jnp.transpose(xyz, (2, 0, 1))
    idx = _fps(xyzT)
    idx4 = idx.reshape(B, _G // _Q, _Q, 1)
    xyzBT = jnp.transpose(xyz, (0, 2, 1))
    knn4, qT5 = _knn(xyzBT, idx4)
    new_xyz = jnp.transpose(qT5.reshape(3, B, _G), (1, 2, 0))

    BR = B * _G * _K
    table = jnp.pad(xyz.reshape(B * N, C), ((0, 0), (0, 16 - C)))
    idx2 = knn4.reshape(BR // 128, 128)
    g = _sc_gather(table, idx2)

    q16 = jnp.pad(new_xyz.reshape(B * _G, C), ((0, 0), (0, 16 - C)))
    R = BR * 16 // 128
    g2 = g.reshape(R, 128)
    q2 = jnp.broadcast_to(
        q16[:, None, None, :], (B * _G, 4, 8, 16)
    ).reshape(R, 128)
    s1, s2 = _sums(g2, q2)
    d2 = _scale(g2, q2, s1, s2)

    knn_xyz = d2.reshape(BR, 16)[:, :C].reshape(B, _G, _K, C)
    tiled = jnp.broadcast_to(
        new_xyz.reshape(B, _G, 1, C), (B, _G, _K, C)
    )
    knn_xyz = jnp.concatenate([knn_xyz, tiled], axis=-1)
    return (new_xyz, knn_xyz)


# speedup vs baseline: 8.0039x; 1.0287x over previous
import functools

import jax
import jax.numpy as jnp
from jax import lax
from jax.experimental import pallas as pl
from jax.experimental.pallas import tpu as pltpu
from jax.experimental.pallas import tpu_sc as plsc

_G = 1024
_K = 32
_INF = 3.0e38


def _fps_body(xyzT_ref, idx_ref, dist_scratch, far_scratch):
    B = xyzT_ref.shape[1]
    N = xyzT_ref.shape[2]
    col = lax.broadcasted_iota(jnp.int32, (B, N), 1)
    gcol = lax.broadcasted_iota(jnp.int32, (B, _G), 1)
    dist_scratch[...] = jnp.full((B, N), 1e10, jnp.float32)
    far_scratch[...] = jnp.zeros((B, 1), jnp.int32)
    idx_ref[...] = jnp.zeros((B, _G), jnp.int32)

    def body(i, _):
        farthest = far_scratch[...]
        idx_ref[...] = jnp.where(gcol == i, farthest, idx_ref[...])
        X = xyzT_ref[0]
        Y = xyzT_ref[1]
        Z = xyzT_ref[2]
        mask = col == farthest
        cx = jnp.sum(jnp.where(mask, X, 0.0), axis=1, keepdims=True)
        cy = jnp.sum(jnp.where(mask, Y, 0.0), axis=1, keepdims=True)
        cz = jnp.sum(jnp.where(mask, Z, 0.0), axis=1, keepdims=True)
        dx = X - cx
        dy = Y - cy
        dz = Z - cz
        dist = dx * dx + dy * dy + dz * dz
        distance = jnp.minimum(dist_scratch[...], dist)
        dist_scratch[...] = distance
        m = jnp.max(distance, axis=1, keepdims=True)
        far_scratch[...] = jnp.min(
            jnp.where(distance == m, col, N), axis=1, keepdims=True
        ).astype(jnp.int32)
        return 0

    lax.fori_loop(0, _G, body, 0)


def _fps(xyzT):
    B, N = xyzT.shape[1], xyzT.shape[2]
    return pl.pallas_call(
        _fps_body,
        out_shape=jax.ShapeDtypeStruct((B, _G), jnp.int32),
        scratch_shapes=[
            pltpu.VMEM((B, N), jnp.float32),
            pltpu.VMEM((B, 1), jnp.int32),
        ],
    )(xyzT)


_Q = 256


def _knn_body(xyzBT_ref, idx_ref, knn_ref, qT_ref):
    N = xyzBT_ref.shape[2]
    col = lax.broadcasted_iota(jnp.int32, (_Q, N), 1)
    kcol = lax.broadcasted_iota(jnp.int32, (_Q, _K), 1)
    qidx = idx_ref[0, 0]
    X = jnp.broadcast_to(xyzBT_ref[0, 0].reshape(1, N), (_Q, N))
    Y = jnp.broadcast_to(xyzBT_ref[0, 1].reshape(1, N), (_Q, N))
    Z = jnp.broadcast_to(xyzBT_ref[0, 2].reshape(1, N), (_Q, N))
    mask = col == qidx
    qx = jnp.sum(jnp.where(mask, X, 0.0), axis=1, keepdims=True)
    qy = jnp.sum(jnp.where(mask, Y, 0.0), axis=1, keepdims=True)
    qz = jnp.sum(jnp.where(mask, Z, 0.0), axis=1, keepdims=True)
    qT_ref[0, 0, 0] = qx
    qT_ref[1, 0, 0] = qy
    qT_ref[2, 0, 0] = qz
    q = jnp.concatenate([qx, qy, qz], axis=1)
    dst = xyzBT_ref[0]
    mm = lax.dot_general(q, dst, (((1,), (0,)), ((), ())),
                         preferred_element_type=jnp.float32)
    nq = jnp.sum(q * q, axis=1, keepdims=True)
    nd = jnp.sum(dst * dst, axis=0, keepdims=True)
    D0 = -2.0 * mm
    D0 = D0 + nq
    D0 = D0 + nd

    knn_ref[0, 0] = jnp.zeros((_Q, _K), jnp.int32)

    def body(k, D):
        m = jnp.min(D, axis=1, keepdims=True)
        sel = D == m
        idxk = jnp.min(jnp.where(sel, col, N), axis=1, keepdims=True)
        knn_ref[0, 0] = jnp.where(kcol == k, idxk, knn_ref[0, 0])
        return jnp.where(col == idxk, _INF, D)

    lax.fori_loop(0, _K, body, D0)
    knn_ref[0, 0] = knn_ref[0, 0] + pl.program_id(0) * N


def _knn(xyzBT, idx4):
    B, N = xyzBT.shape[0], xyzBT.shape[2]
    QB = _G // _Q
    grid = (B, QB)
    return pl.pallas_call(
        _knn_body,
        grid=grid,
        in_specs=[
            pl.BlockSpec((1, 3, N), lambda b, q: (b, 0, 0)),
            pl.BlockSpec((1, 1, _Q, 1), lambda b, q: (b, q, 0, 0)),
        ],
        out_specs=[
            pl.BlockSpec((1, 1, _Q, _K), lambda b, q: (b, q, 0, 0)),
            pl.BlockSpec((3, 1, 1, _Q, 1), lambda b, q: (0, b, q, 0, 0)),
        ],
        out_shape=[
            jax.ShapeDtypeStruct((B, QB, _Q, _K), jnp.int32),
            jax.ShapeDtypeStruct((3, B, QB, _Q, 1), jnp.float32),
        ],
        compiler_params=pltpu.CompilerParams(
            dimension_semantics=("arbitrary", "arbitrary"),
        ),
    )(xyzBT, idx4)


_NC, _NS = 2, 16
_NW = _NC * _NS


def _sc_gather(table, idx2):
    BR = idx2.shape[0] * 128
    rows_per_w = BR // _NW
    n_chunks = rows_per_w // 128
    mesh = plsc.VectorSubcoreMesh(core_axis_name="c", subcore_axis_name="s")

    @functools.partial(
        pl.kernel,
        mesh=mesh,
        out_type=jax.ShapeDtypeStruct((BR, 16), jnp.float32),
        compiler_params=pltpu.CompilerParams(use_tc_tiling_on_sc=False),
        scratch_types=[
            pltpu.VMEM((n_chunks, 128), jnp.int32),
            pltpu.VMEM((128, 16), jnp.float32),
            pltpu.VMEM((128, 16), jnp.float32),
            pltpu.SemaphoreType.DMA,
            pltpu.SemaphoreType.DMA,
        ],
    )
    def k(table_hbm, idx_hbm, out_hbm, idx_v, buf0, buf1, sem0, sem1):
        wid = lax.axis_index("s") * _NC + lax.axis_index("c")
        base = wid * rows_per_w
        pltpu.sync_copy(idx_hbm.at[pl.ds(wid * n_chunks, n_chunks)], idx_v)
        bufs = (buf0, buf1)
        sems = (sem0, sem1)
        handles = [
            pltpu.async_copy(table_hbm.at[idx_v.at[0]], buf0, sem0),
            pltpu.async_copy(table_hbm.at[idx_v.at[1]], buf1, sem1),
        ]
        for c in range(n_chunks):
            b = c % 2
            handles[b].wait()
            pltpu.sync_copy(bufs[b], out_hbm.at[pl.ds(base + c * 128, 128)])
            if c + 2 < n_chunks:
                handles[b] = pltpu.async_copy(
                    table_hbm.at[idx_v.at[c + 2]], bufs[b], sems[b]
                )

    return k(table, idx2)


_NBLK = 32


def _sums_body(g_ref, q_ref, s1_ref, s2_ref):
    d = g_ref[...] - q_ref[...]
    s1_ref[0] = jnp.sum(d, axis=0, keepdims=True)
    s2_ref[0] = jnp.sum(d * d, axis=0, keepdims=True)


def _sums(g2, q2):
    R = g2.shape[0]
    blk = R // _NBLK
    return pl.pallas_call(
        _sums_body,
        grid=(_NBLK,),
        in_specs=[
            pl.BlockSpec((blk, 128), lambda c: (c, 0)),
            pl.BlockSpec((blk, 128), lambda c: (c, 0)),
        ],
        out_specs=[
            pl.BlockSpec((1, 1, 128), lambda c: (c, 0, 0)),
            pl.BlockSpec((1, 1, 128), lambda c: (c, 0, 0)),
        ],
        out_shape=[
            jax.ShapeDtypeStruct((_NBLK, 1, 128), jnp.float32),
            jax.ShapeDtypeStruct((_NBLK, 1, 128), jnp.float32),
        ],
    )(g2, q2)


_M = 8 * _G * _K * 3


def _scale_body(g_ref, q_ref, s1_ref, s2_ref, out_ref):
    S1 = jnp.sum(s1_ref[...])
    S2 = jnp.sum(s2_ref[...])
    var = (S2 - S1 * S1 / _M) / (_M - 1)
    std = jnp.sqrt(var)
    out_ref[...] = (g_ref[...] - q_ref[...]) / (std + 1e-5)


def _scale(g2, q2, s1, s2):
    R = g2.shape[0]
    blk = R // _NBLK
    return pl.pallas_call(
        _scale_body,
        grid=(_NBLK,),
        in_specs=[
            pl.BlockSpec((blk, 128), lambda c: (c, 0)),
            pl.BlockSpec((blk, 128), lambda c: (c, 0)),
            pl.BlockSpec((_NBLK, 1, 128), lambda c: (0, 0, 0)),
            pl.BlockSpec((_NBLK, 1, 128), lambda c: (0, 0, 0)),
        ],
        out_specs=pl.BlockSpec((blk, 128), lambda c: (c, 0)),
        out_shape=jax.ShapeDtypeStruct((R, 128), jnp.float32),
    )(g2, q2, s1, s2)


def kernel(xyz):
    B, N, C = xyz.shape
    xyzT = jnp.transpose(xyz, (2, 0, 1))
    idx = _fps(xyzT)
    idx4 = idx.reshape(B, _G // _Q, _Q, 1)
    xyzBT = jnp.transpose(xyz, (0, 2, 1))
    knn4, qT5 = _knn(xyzBT, idx4)
    new_xyz = jnp.transpose(qT5.reshape(3, B, _G), (1, 2, 0))

    BR = B * _G * _K
    table = jnp.pad(xyz.reshape(B * N, C), ((0, 0), (0, 16 - C)))
    idx2 = knn4.reshape(BR // 128, 128)
    g = _sc_gather(table, idx2)

    q16 = jnp.pad(new_xyz.reshape(B * _G, C), ((0, 0), (0, 16 - C)))
    R = BR * 16 // 128
    g2 = g.reshape(R, 128)
    q2 = jnp.broadcast_to(
        q16[:, None, None, :], (B * _G, 4, 8, 16)
    ).reshape(R, 128)
    s1, s2 = _sums(g2, q2)
    d2 = _scale(g2, q2, s1, s2)

    knn_xyz = d2.reshape(BR, 16)[:, :C].reshape(B, _G, _K, C)
    tiled = jnp.broadcast_to(
        new_xyz.reshape(B, _G, 1, C), (B, _G, _K, C)
    )
    knn_xyz = jnp.concatenate([knn_xyz, tiled], axis=-1)
    return (new_xyz, knn_xyz)


# speedup vs baseline: 8.3657x; 1.0452x over previous
import functools

import jax
import jax.numpy as jnp
from jax import lax
from jax.experimental import pallas as pl
from jax.experimental.pallas import tpu as pltpu
from jax.experimental.pallas import tpu_sc as plsc

_G = 1024
_K = 32
_INF = 3.0e38


def _fps_body(xyzT_ref, idx_ref, dist_scratch, far_scratch):
    B = xyzT_ref.shape[1]
    N = xyzT_ref.shape[2]
    col = lax.broadcasted_iota(jnp.int32, (B, N), 1)
    gcol = lax.broadcasted_iota(jnp.int32, (B, _G), 1)
    dist_scratch[...] = jnp.full((B, N), 1e10, jnp.float32)
    far_scratch[...] = jnp.zeros((B, 1), jnp.int32)
    idx_ref[...] = jnp.zeros((B, _G), jnp.int32)

    def body(i, _):
        farthest = far_scratch[...]
        idx_ref[...] = jnp.where(gcol == i, farthest, idx_ref[...])
        X = xyzT_ref[0]
        Y = xyzT_ref[1]
        Z = xyzT_ref[2]
        mask = col == farthest
        cx = jnp.sum(jnp.where(mask, X, 0.0), axis=1, keepdims=True)
        cy = jnp.sum(jnp.where(mask, Y, 0.0), axis=1, keepdims=True)
        cz = jnp.sum(jnp.where(mask, Z, 0.0), axis=1, keepdims=True)
        dx = X - cx
        dy = Y - cy
        dz = Z - cz
        dist = dx * dx + dy * dy + dz * dz
        distance = jnp.minimum(dist_scratch[...], dist)
        dist_scratch[...] = distance
        m = jnp.max(distance, axis=1, keepdims=True)
        far_scratch[...] = jnp.min(
            jnp.where(distance == m, col, N), axis=1, keepdims=True
        ).astype(jnp.int32)
        return 0

    lax.fori_loop(0, _G, body, 0)


def _fps(xyzT):
    B, N = xyzT.shape[1], xyzT.shape[2]
    return pl.pallas_call(
        _fps_body,
        out_shape=jax.ShapeDtypeStruct((B, _G), jnp.int32),
        scratch_shapes=[
            pltpu.VMEM((B, N), jnp.float32),
            pltpu.VMEM((B, 1), jnp.int32),
        ],
    )(xyzT)


_Q = 256


def _tree_min(x):
    n = x.shape[1]
    while n > 256:
        x = jnp.minimum(x[:, : n // 2], x[:, n // 2 :])
        n //= 2
    return jnp.min(x, axis=1, keepdims=True)


def _tree_max(x):
    n = x.shape[1]
    while n > 256:
        x = jnp.maximum(x[:, : n // 2], x[:, n // 2 :])
        n //= 2
    return jnp.max(x, axis=1, keepdims=True)


def _knn_body(xyzBT_ref, idx_ref, knn_ref, qT_ref):
    N = xyzBT_ref.shape[2]
    col = lax.broadcasted_iota(jnp.int32, (_Q, N), 1)
    kcol = lax.broadcasted_iota(jnp.int32, (_Q, _K), 1)
    qidx = idx_ref[0, 0]
    X = jnp.broadcast_to(xyzBT_ref[0, 0].reshape(1, N), (_Q, N))
    Y = jnp.broadcast_to(xyzBT_ref[0, 1].reshape(1, N), (_Q, N))
    Z = jnp.broadcast_to(xyzBT_ref[0, 2].reshape(1, N), (_Q, N))
    mask = col == qidx
    qx = jnp.sum(jnp.where(mask, X, 0.0), axis=1, keepdims=True)
    qy = jnp.sum(jnp.where(mask, Y, 0.0), axis=1, keepdims=True)
    qz = jnp.sum(jnp.where(mask, Z, 0.0), axis=1, keepdims=True)
    qT_ref[0, 0, 0] = qx
    qT_ref[1, 0, 0] = qy
    qT_ref[2, 0, 0] = qz
    q = jnp.concatenate([qx, qy, qz], axis=1)
    dst = xyzBT_ref[0]
    mm = lax.dot_general(q, dst, (((1,), (0,)), ((), ())),
                         preferred_element_type=jnp.float32)
    nq = jnp.sum(q * q, axis=1, keepdims=True)
    nd = jnp.sum(dst * dst, axis=0, keepdims=True)
    D0 = -2.0 * mm
    D0 = D0 + nq
    D0 = D0 + nd

    knn_ref[0, 0] = jnp.zeros((_Q, _K), jnp.int32)

    def body(k, D):
        m = _tree_min(D)
        sel = D == m
        idxk = _tree_min(jnp.where(sel, col, N))
        knn_ref[0, 0] = jnp.where(kcol == k, idxk, knn_ref[0, 0])
        return jnp.where(col == idxk, _INF, D)

    lax.fori_loop(0, _K, body, D0)
    knn_ref[0, 0] = knn_ref[0, 0] + pl.program_id(0) * N


def _knn(xyzBT, idx4):
    B, N = xyzBT.shape[0], xyzBT.shape[2]
    QB = _G // _Q
    grid = (B, QB)
    return pl.pallas_call(
        _knn_body,
        grid=grid,
        in_specs=[
            pl.BlockSpec((1, 3, N), lambda b, q: (b, 0, 0)),
            pl.BlockSpec((1, 1, _Q, 1), lambda b, q: (b, q, 0, 0)),
        ],
        out_specs=[
            pl.BlockSpec((1, 1, _Q, _K), lambda b, q: (b, q, 0, 0)),
            pl.BlockSpec((3, 1, 1, _Q, 1), lambda b, q: (0, b, q, 0, 0)),
        ],
        out_shape=[
            jax.ShapeDtypeStruct((B, QB, _Q, _K), jnp.int32),
            jax.ShapeDtypeStruct((3, B, QB, _Q, 1), jnp.float32),
        ],
        compiler_params=pltpu.CompilerParams(
            dimension_semantics=("arbitrary", "arbitrary"),
        ),
    )(xyzBT, idx4)


_NC, _NS = 2, 16
_NW = _NC * _NS


def _sc_gather(table, idx2):
    BR = idx2.shape[0] * 128
    rows_per_w = BR // _NW
    n_chunks = rows_per_w // 128
    mesh = plsc.VectorSubcoreMesh(core_axis_name="c", subcore_axis_name="s")

    @functools.partial(
        pl.kernel,
        mesh=mesh,
        out_type=jax.ShapeDtypeStruct((BR, 16), jnp.float32),
        compiler_params=pltpu.CompilerParams(use_tc_tiling_on_sc=False),
        scratch_types=[
            pltpu.VMEM((n_chunks, 128), jnp.int32),
            pltpu.VMEM((128, 16), jnp.float32),
            pltpu.VMEM((128, 16), jnp.float32),
            pltpu.SemaphoreType.DMA,
            pltpu.SemaphoreType.DMA,
        ],
    )
    def k(table_hbm, idx_hbm, out_hbm, idx_v, buf0, buf1, sem0, sem1):
        wid = lax.axis_index("s") * _NC + lax.axis_index("c")
        base = wid * rows_per_w
        pltpu.sync_copy(idx_hbm.at[pl.ds(wid * n_chunks, n_chunks)], idx_v)
        bufs = (buf0, buf1)
        sems = (sem0, sem1)
        handles = [
            pltpu.async_copy(table_hbm.at[idx_v.at[0]], buf0, sem0),
            pltpu.async_copy(table_hbm.at[idx_v.at[1]], buf1, sem1),
        ]
        for c in range(n_chunks):
            b = c % 2
            handles[b].wait()
            pltpu.sync_copy(bufs[b], out_hbm.at[pl.ds(base + c * 128, 128)])
            if c + 2 < n_chunks:
                handles[b] = pltpu.async_copy(
                    table_hbm.at[idx_v.at[c + 2]], bufs[b], sems[b]
                )

    return k(table, idx2)


_NBLK = 32


def _sums_body(g_ref, q_ref, s1_ref, s2_ref):
    d = g_ref[...] - q_ref[...]
    s1_ref[0] = jnp.sum(d, axis=0, keepdims=True)
    s2_ref[0] = jnp.sum(d * d, axis=0, keepdims=True)


def _sums(g2, q2):
    R = g2.shape[0]
    blk = R // _NBLK
    return pl.pallas_call(
        _sums_body,
        grid=(_NBLK,),
        in_specs=[
            pl.BlockSpec((blk, 128), lambda c: (c, 0)),
            pl.BlockSpec((blk, 128), lambda c: (c, 0)),
        ],
        out_specs=[
            pl.BlockSpec((1, 1, 128), lambda c: (c, 0, 0)),
            pl.BlockSpec((1, 1, 128), lambda c: (c, 0, 0)),
        ],
        out_shape=[
            jax.ShapeDtypeStruct((_NBLK, 1, 128), jnp.float32),
            jax.ShapeDtypeStruct((_NBLK, 1, 128), jnp.float32),
        ],
    )(g2, q2)


_M = 8 * _G * _K * 3


def _scale_body(g_ref, q_ref, s1_ref, s2_ref, out_ref):
    S1 = jnp.sum(s1_ref[...])
    S2 = jnp.sum(s2_ref[...])
    var = (S2 - S1 * S1 / _M) / (_M - 1)
    std = jnp.sqrt(var)
    out_ref[...] = (g_ref[...] - q_ref[...]) / (std + 1e-5)


def _scale(g2, q2, s1, s2):
    R = g2.shape[0]
    blk = R // _NBLK
    return pl.pallas_call(
        _scale_body,
        grid=(_NBLK,),
        in_specs=[
            pl.BlockSpec((blk, 128), lambda c: (c, 0)),
            pl.BlockSpec((blk, 128), lambda c: (c, 0)),
            pl.BlockSpec((_NBLK, 1, 128), lambda c: (0, 0, 0)),
            pl.BlockSpec((_NBLK, 1, 128), lambda c: (0, 0, 0)),
        ],
        out_specs=pl.BlockSpec((blk, 128), lambda c: (c, 0)),
        out_shape=jax.ShapeDtypeStruct((R, 128), jnp.float32),
    )(g2, q2, s1, s2)


def kernel(xyz):
    B, N, C = xyz.shape
    xyzT = jnp.transpose(xyz, (2, 0, 1))
    idx = _fps(xyzT)
    idx4 = idx.reshape(B, _G // _Q, _Q, 1)
    xyzBT = jnp.transpose(xyz, (0, 2, 1))
    knn4, qT5 = _knn(xyzBT, idx4)
    new_xyz = jnp.transpose(qT5.reshape(3, B, _G), (1, 2, 0))

    BR = B * _G * _K
    table = jnp.pad(xyz.reshape(B * N, C), ((0, 0), (0, 16 - C)))
    idx2 = knn4.reshape(BR // 128, 128)
    g = _sc_gather(table, idx2)

    q16 = jnp.pad(new_xyz.reshape(B * _G, C), ((0, 0), (0, 16 - C)))
    R = BR * 16 // 128
    g2 = g.reshape(R, 128)
    q2 = jnp.broadcast_to(
        q16[:, None, None, :], (B * _G, 4, 8, 16)
    ).reshape(R, 128)
    s1, s2 = _sums(g2, q2)
    d2 = _scale(g2, q2, s1, s2)

    knn_xyz = d2.reshape(BR, 16)[:, :C].reshape(B, _G, _K, C)
    tiled = jnp.broadcast_to(
        new_xyz.reshape(B, _G, 1, C), (B, _G, _K, C)
    )
    knn_xyz = jnp.concatenate([knn_xyz, tiled], axis=-1)
    return (new_xyz, knn_xyz)
